# baseline (speedup 1.0000x reference)
import functools
import jax
import jax.numpy as jnp
from jax.experimental import pallas as pl
from jax.experimental.pallas import tpu as pltpu

K = 3
BM = 256


def _fused_body(ps_ref, posT_ref, x_ref, w1_ref, b1_ref, w2_ref, b2_ref,
                out_ref):
    n = posT_ref.shape[1]
    bm = ps_ref.shape[0]

    ps = ps_ref[...]
    posT = posT_ref[...]
    qp = jnp.dot(ps, posT, preferred_element_type=jnp.float32,
                 precision=jax.lax.Precision.HIGHEST)
    pn = jnp.sum(posT * posT, axis=0, keepdims=True)
    qn = jnp.sum(ps * ps, axis=1, keepdims=True)
    e = pn - 2.0 * qp

    s = jnp.zeros((bm, n), dtype=jnp.float32)
    wsum = jnp.zeros((bm, 1), dtype=jnp.float32)
    for k in range(K):
        m_e = jnp.min(e, axis=1, keepdims=True)
        m_d = m_e + qn
        w_k = 1.0 / jnp.maximum(m_d, 1e-16)
        hit = e == m_e
        s = jnp.where(hit, w_k, s)
        if k < K - 1:
            e = jnp.where(hit, jnp.inf, e)
        wsum = wsum + w_k

    interp = jnp.dot(s, x_ref[...], preferred_element_type=jnp.float32,
                     precision=jax.lax.Precision.HIGHEST)
    interp = interp / wsum

    h1 = jnp.dot(interp, w1_ref[...], preferred_element_type=jnp.float32)
    h1 = jnp.maximum(h1 + b1_ref[...], 0.0)
    h2 = jnp.dot(h1, w2_ref[...], preferred_element_type=jnp.float32)
    out_ref[...] = h2 + b2_ref[...]


def kernel(x, pos, x_skip, pos_skip, assign_index, W1, b1, W2, b2):
    del x_skip, assign_index
    n, d_feat = x.shape
    m = pos_skip.shape[0]
    h_feat = W2.shape[1]

    posT = pos.T
    b1_2d = b1.reshape(1, -1)
    b2_2d = b2.reshape(1, -1)

    grid = (m // BM,)
    out = pl.pallas_call(
        _fused_body,
        grid=grid,
        in_specs=[
            pl.BlockSpec((BM, 3), lambda i: (i, 0)),
            pl.BlockSpec((3, n), lambda i: (0, 0)),
            pl.BlockSpec((n, d_feat), lambda i: (0, 0)),
            pl.BlockSpec((d_feat, h_feat), lambda i: (0, 0)),
            pl.BlockSpec((1, h_feat), lambda i: (0, 0)),
            pl.BlockSpec((h_feat, h_feat), lambda i: (0, 0)),
            pl.BlockSpec((1, h_feat), lambda i: (0, 0)),
        ],
        out_specs=pl.BlockSpec((BM, h_feat), lambda i: (i, 0)),
        out_shape=jax.ShapeDtypeStruct((m, h_feat), jnp.float32),
    )(pos_skip, posT, x, W1, b1_2d, W2, b2_2d)

    return (out, pos_skip)


# speedup vs baseline: 1.9906x; 1.9906x over previous
import functools
import jax
import jax.numpy as jnp
from jax.experimental import pallas as pl
from jax.experimental.pallas import tpu as pltpu

K = 3
BM = 256


def _fused_body(ps_ref, posT_ref, x_ref, w1_ref, b1_ref, w2_ref, b2_ref,
                out_ref):
    n = posT_ref.shape[1]
    bm = ps_ref.shape[0]

    ps = ps_ref[...]
    posT = posT_ref[...]
    qp = jnp.dot(ps, posT, preferred_element_type=jnp.float32,
                 precision=jax.lax.Precision.HIGHEST)
    pn = jnp.sum(posT * posT, axis=0, keepdims=True)
    qn = jnp.sum(ps * ps, axis=1, keepdims=True)
    e = pn - 2.0 * qp

    s = jnp.zeros((bm, n), dtype=jnp.float32)
    wsum = jnp.zeros((bm, 1), dtype=jnp.float32)
    for k in range(K):
        m_e = jnp.min(e, axis=1, keepdims=True)
        m_d = m_e + qn
        w_k = 1.0 / jnp.maximum(m_d, 1e-16)
        hit = e == m_e
        s = jnp.where(hit, w_k, s)
        if k < K - 1:
            e = jnp.where(hit, jnp.inf, e)
        wsum = wsum + w_k

    interp = jnp.dot(s, x_ref[...], preferred_element_type=jnp.float32)
    interp = interp / wsum

    h1 = jnp.dot(interp, w1_ref[...], preferred_element_type=jnp.float32)
    h1 = jnp.maximum(h1 + b1_ref[...], 0.0)
    h2 = jnp.dot(h1, w2_ref[...], preferred_element_type=jnp.float32)
    out_ref[...] = h2 + b2_ref[...]


def kernel(x, pos, x_skip, pos_skip, assign_index, W1, b1, W2, b2):
    del x_skip, assign_index
    n, d_feat = x.shape
    m = pos_skip.shape[0]
    h_feat = W2.shape[1]

    posT = pos.T
    b1_2d = b1.reshape(1, -1)
    b2_2d = b2.reshape(1, -1)

    grid = (m // BM,)
    out = pl.pallas_call(
        _fused_body,
        grid=grid,
        in_specs=[
            pl.BlockSpec((BM, 3), lambda i: (i, 0)),
            pl.BlockSpec((3, n), lambda i: (0, 0)),
            pl.BlockSpec((n, d_feat), lambda i: (0, 0)),
            pl.BlockSpec((d_feat, h_feat), lambda i: (0, 0)),
            pl.BlockSpec((1, h_feat), lambda i: (0, 0)),
            pl.BlockSpec((h_feat, h_feat), lambda i: (0, 0)),
            pl.BlockSpec((1, h_feat), lambda i: (0, 0)),
        ],
        out_specs=pl.BlockSpec((BM, h_feat), lambda i: (i, 0)),
        out_shape=jax.ShapeDtypeStruct((m, h_feat), jnp.float32),
    )(pos_skip, posT, x, W1, b1_2d, W2, b2_2d)

    return (out, pos_skip)


# speedup vs baseline: 2.9572x; 1.4856x over previous
import jax
import jax.numpy as jnp
from jax.experimental import pallas as pl

K = 3
BM = 256


def _fused_body(ps_ref, posT_ref, x_ref, w1_ref, b1_ref, w2_ref, b2_ref,
                out_ref):
    n = posT_ref.shape[1]
    bm = ps_ref.shape[1]

    d = jnp.zeros((bm, n), dtype=jnp.float32)
    for c in range(3):
        q_c = ps_ref[c, :].reshape(bm, 1)
        p_c = posT_ref[c, :].reshape(1, n)
        diff = q_c - p_c
        d = d + diff * diff

    s = jnp.zeros((bm, n), dtype=jnp.float32)
    wsum = jnp.zeros((bm, 1), dtype=jnp.float32)
    for k in range(K):
        m = jnp.min(d, axis=1, keepdims=True)
        w_k = 1.0 / jnp.maximum(m, 1e-16)
        hit = d == m
        s = jnp.where(hit, w_k, s)
        if k < K - 1:
            d = jnp.where(hit, jnp.inf, d)
        wsum = wsum + w_k

    interp = jnp.dot(s, x_ref[...], preferred_element_type=jnp.float32)
    interp = interp / wsum

    h1 = jnp.dot(interp, w1_ref[...], preferred_element_type=jnp.float32)
    h1 = jnp.maximum(h1 + b1_ref[...], 0.0)
    h2 = jnp.dot(h1, w2_ref[...], preferred_element_type=jnp.float32)
    out_ref[...] = h2 + b2_ref[...]


def kernel(x, pos, x_skip, pos_skip, assign_index, W1, b1, W2, b2):
    del x_skip, assign_index
    n, d_feat = x.shape
    m = pos_skip.shape[0]
    h_feat = W2.shape[1]

    posT = pos.T
    pos_skipT = pos_skip.T
    x_bf = x.astype(jnp.bfloat16)
    w1_bf = W1.astype(jnp.bfloat16)
    w2_bf = W2.astype(jnp.bfloat16)
    b1_2d = b1.reshape(1, -1)
    b2_2d = b2.reshape(1, -1)

    grid = (m // BM,)
    out = pl.pallas_call(
        _fused_body,
        grid=grid,
        in_specs=[
            pl.BlockSpec((3, BM), lambda i: (0, i)),
            pl.BlockSpec((3, n), lambda i: (0, 0)),
            pl.BlockSpec((n, d_feat), lambda i: (0, 0)),
            pl.BlockSpec((d_feat, h_feat), lambda i: (0, 0)),
            pl.BlockSpec((1, h_feat), lambda i: (0, 0)),
            pl.BlockSpec((h_feat, h_feat), lambda i: (0, 0)),
            pl.BlockSpec((1, h_feat), lambda i: (0, 0)),
        ],
        out_specs=pl.BlockSpec((BM, h_feat), lambda i: (i, 0)),
        out_shape=jax.ShapeDtypeStruct((m, h_feat), jnp.float32),
    )(pos_skipT, posT, x_bf, w1_bf, b1_2d, w2_bf, b2_2d)

    return (out, pos_skip)


# speedup vs baseline: 3.2346x; 1.0938x over previous
import jax
import jax.numpy as jnp
from jax.experimental import pallas as pl

K = 3
BM = 512


def _fused_body(ps_ref, posT_ref, x_ref, w1_ref, b1_ref, w2_ref, b2_ref,
                out_ref):
    n = posT_ref.shape[1]
    bm = ps_ref.shape[1]

    d = jnp.zeros((bm, n), dtype=jnp.float32)
    for c in range(3):
        q_c = ps_ref[c, :].reshape(bm, 1)
        p_c = posT_ref[c, :].reshape(1, n)
        diff = q_c - p_c
        d = d + diff * diff

    s = jnp.zeros((bm, n), dtype=jnp.float32)
    wsum = jnp.zeros((bm, 1), dtype=jnp.float32)
    for k in range(K):
        m = jnp.min(d, axis=1, keepdims=True)
        w_k = 1.0 / jnp.maximum(m, 1e-16)
        hit = d == m
        s = jnp.where(hit, w_k, s)
        if k < K - 1:
            d = jnp.where(hit, jnp.inf, d)
        wsum = wsum + w_k

    interp = jnp.dot(s, x_ref[...], preferred_element_type=jnp.float32)
    interp = interp / wsum

    h1 = jnp.dot(interp, w1_ref[...], preferred_element_type=jnp.float32)
    h1 = jnp.maximum(h1 + b1_ref[...], 0.0)
    h2 = jnp.dot(h1, w2_ref[...], preferred_element_type=jnp.float32)
    out_ref[...] = h2 + b2_ref[...]


def kernel(x, pos, x_skip, pos_skip, assign_index, W1, b1, W2, b2):
    del x_skip, assign_index
    n, d_feat = x.shape
    m = pos_skip.shape[0]
    h_feat = W2.shape[1]

    posT = pos.T
    pos_skipT = pos_skip.T
    x_bf = x.astype(jnp.bfloat16)
    w1_bf = W1.astype(jnp.bfloat16)
    w2_bf = W2.astype(jnp.bfloat16)
    b1_2d = b1.reshape(1, -1)
    b2_2d = b2.reshape(1, -1)

    grid = (m // BM,)
    out = pl.pallas_call(
        _fused_body,
        grid=grid,
        in_specs=[
            pl.BlockSpec((3, BM), lambda i: (0, i)),
            pl.BlockSpec((3, n), lambda i: (0, 0)),
            pl.BlockSpec((n, d_feat), lambda i: (0, 0)),
            pl.BlockSpec((d_feat, h_feat), lambda i: (0, 0)),
            pl.BlockSpec((1, h_feat), lambda i: (0, 0)),
            pl.BlockSpec((h_feat, h_feat), lambda i: (0, 0)),
            pl.BlockSpec((1, h_feat), lambda i: (0, 0)),
        ],
        out_specs=pl.BlockSpec((BM, h_feat), lambda i: (i, 0)),
        out_shape=jax.ShapeDtypeStruct((m, h_feat), jnp.float32),
    )(pos_skipT, posT, x_bf, w1_bf, b1_2d, w2_bf, b2_2d)

    return (out, pos_skip)


# speedup vs baseline: 3.2966x; 1.0192x over previous
import jax
import jax.numpy as jnp
from jax.experimental import pallas as pl

K = 3
BM = 1024


def _fused_body(ps_ref, posT_ref, x_ref, w1_ref, b1_ref, w2_ref, b2_ref,
                out_ref):
    n = posT_ref.shape[1]
    bm = ps_ref.shape[1]

    d = jnp.zeros((bm, n), dtype=jnp.float32)
    for c in range(3):
        q_c = ps_ref[c, :].reshape(bm, 1)
        p_c = posT_ref[c, :].reshape(1, n)
        diff = q_c - p_c
        d = d + diff * diff

    s = jnp.zeros((bm, n), dtype=jnp.float32)
    wsum = jnp.zeros((bm, 1), dtype=jnp.float32)
    for k in range(K):
        m = jnp.min(d, axis=1, keepdims=True)
        w_k = 1.0 / jnp.maximum(m, 1e-16)
        hit = d == m
        s = jnp.where(hit, w_k, s)
        if k < K - 1:
            d = jnp.where(hit, jnp.inf, d)
        wsum = wsum + w_k

    interp = jnp.dot(s, x_ref[...], preferred_element_type=jnp.float32)
    interp = interp / wsum

    h1 = jnp.dot(interp, w1_ref[...], preferred_element_type=jnp.float32)
    h1 = jnp.maximum(h1 + b1_ref[...], 0.0)
    h2 = jnp.dot(h1, w2_ref[...], preferred_element_type=jnp.float32)
    out_ref[...] = h2 + b2_ref[...]


def kernel(x, pos, x_skip, pos_skip, assign_index, W1, b1, W2, b2):
    del x_skip, assign_index
    n, d_feat = x.shape
    m = pos_skip.shape[0]
    h_feat = W2.shape[1]

    posT = pos.T
    pos_skipT = pos_skip.T
    x_bf = x.astype(jnp.bfloat16)
    w1_bf = W1.astype(jnp.bfloat16)
    w2_bf = W2.astype(jnp.bfloat16)
    b1_2d = b1.reshape(1, -1)
    b2_2d = b2.reshape(1, -1)

    grid = (m // BM,)
    out = pl.pallas_call(
        _fused_body,
        grid=grid,
        in_specs=[
            pl.BlockSpec((3, BM), lambda i: (0, i)),
            pl.BlockSpec((3, n), lambda i: (0, 0)),
            pl.BlockSpec((n, d_feat), lambda i: (0, 0)),
            pl.BlockSpec((d_feat, h_feat), lambda i: (0, 0)),
            pl.BlockSpec((1, h_feat), lambda i: (0, 0)),
            pl.BlockSpec((h_feat, h_feat), lambda i: (0, 0)),
            pl.BlockSpec((1, h_feat), lambda i: (0, 0)),
        ],
        out_specs=pl.BlockSpec((BM, h_feat), lambda i: (i, 0)),
        out_shape=jax.ShapeDtypeStruct((m, h_feat), jnp.float32),
    )(pos_skipT, posT, x_bf, w1_bf, b1_2d, w2_bf, b2_2d)

    return (out, pos_skip)


# speedup vs baseline: 3.6069x; 1.0941x over previous
import jax
import jax.numpy as jnp
from jax.experimental import pallas as pl

K = 3
BM = 1024


def _fused_body(ps_ref, posT_ref, x_ref, w1_ref, b1_ref, w2_ref, b2_ref,
                out_ref):
    n = posT_ref.shape[1]
    bm = ps_ref.shape[1]

    pn = jnp.sum(posT_ref[...] * posT_ref[...], axis=0, keepdims=True)
    e = pn
    qn = jnp.zeros((bm, 1), dtype=jnp.float32)
    for c in range(3):
        q_c = ps_ref[c, :].reshape(bm, 1)
        p_c = posT_ref[c, :].reshape(1, n)
        e = e + q_c * (-2.0 * p_c)
        qn = qn + q_c * q_c

    s = jnp.zeros((bm, n), dtype=jnp.float32)
    wsum = jnp.zeros((bm, 1), dtype=jnp.float32)
    for k in range(K):
        m_e = jnp.min(e, axis=1, keepdims=True)
        w_k = 1.0 / jnp.maximum(m_e + qn, 1e-16)
        hit = e == m_e
        s = jnp.where(hit, w_k, s)
        if k < K - 1:
            e = jnp.where(hit, jnp.inf, e)
        wsum = wsum + w_k

    interp = jnp.dot(s, x_ref[...], preferred_element_type=jnp.float32)
    interp = interp / wsum

    h1 = jnp.dot(interp, w1_ref[...], preferred_element_type=jnp.float32)
    h1 = jnp.maximum(h1 + b1_ref[...], 0.0)
    h2 = jnp.dot(h1, w2_ref[...], preferred_element_type=jnp.float32)
    out_ref[...] = h2 + b2_ref[...]


def kernel(x, pos, x_skip, pos_skip, assign_index, W1, b1, W2, b2):
    del x_skip, assign_index
    n, d_feat = x.shape
    m = pos_skip.shape[0]
    h_feat = W2.shape[1]

    posT = pos.T
    pos_skipT = pos_skip.T
    x_bf = x.astype(jnp.bfloat16)
    w1_bf = W1.astype(jnp.bfloat16)
    w2_bf = W2.astype(jnp.bfloat16)
    b1_2d = b1.reshape(1, -1)
    b2_2d = b2.reshape(1, -1)

    grid = (m // BM,)
    out = pl.pallas_call(
        _fused_body,
        grid=grid,
        in_specs=[
            pl.BlockSpec((3, BM), lambda i: (0, i)),
            pl.BlockSpec((3, n), lambda i: (0, 0)),
            pl.BlockSpec((n, d_feat), lambda i: (0, 0)),
            pl.BlockSpec((d_feat, h_feat), lambda i: (0, 0)),
            pl.BlockSpec((1, h_feat), lambda i: (0, 0)),
            pl.BlockSpec((h_feat, h_feat), lambda i: (0, 0)),
            pl.BlockSpec((1, h_feat), lambda i: (0, 0)),
        ],
        out_specs=pl.BlockSpec((BM, h_feat), lambda i: (i, 0)),
        out_shape=jax.ShapeDtypeStruct((m, h_feat), jnp.float32),
    )(pos_skipT, posT, x_bf, w1_bf, b1_2d, w2_bf, b2_2d)

    return (out, pos_skip)
